# baseline (speedup 1.0000x reference)
import functools

import jax
import jax.numpy as jnp
from jax import lax
from jax.experimental import pallas as pl
from jax.experimental.pallas import tpu as pltpu
from jax.experimental.pallas import tpu_sc as plsc

B, H, W_ = 4, 384, 384
CIN, CF, NCLS = 3, 96, 10
MAXLEN = 512
NTOK = 512
N = H * W_
P = 8192
NT = N // P

SC_NC, SC_NS = 2, 16
NWORK = SC_NC * SC_NS
CHUNK = B * N // NWORK
HBINS = NTOK * NCLS


def _sc_hist_kernel(seg_hbm, gts_hbm, out_hbm, seg_v, gts_v, hist_v):
    wid = lax.axis_index("s") * SC_NC + lax.axis_index("c")
    base = wid * CHUNK
    pltpu.sync_copy(seg_hbm.at[pl.ds(base, CHUNK)], seg_v)
    pltpu.sync_copy(gts_hbm.at[pl.ds(base, CHUNK)], gts_v)

    zeros = jnp.zeros((16,), jnp.float32)

    def zbody(i, carry):
        hist_v[pl.ds(i * 16, 16)] = zeros
        return carry

    lax.fori_loop(0, HBINS, zbody, 0)

    ones = jnp.ones((16,), jnp.float32)
    lanes = lax.iota(jnp.int32, 16) * HBINS

    def body(i, carry):
        s = seg_v[pl.ds(i * 16, 16)]
        g = gts_v[pl.ds(i * 16, 16)]
        idx = lanes + s * NCLS + g
        plsc.addupdate_scatter(hist_v, [idx], ones)
        return carry

    lax.fori_loop(0, CHUNK // 16, body, 0)

    def rbody(j, carry):
        acc = hist_v[pl.ds(j * 16, 16)]
        for l in range(1, 16):
            acc = acc + hist_v[pl.ds(l * HBINS + j * 16, 16)]
        hist_v[pl.ds(j * 16, 16)] = acc
        return carry

    lax.fori_loop(0, HBINS // 16, rbody, 0)
    pltpu.sync_copy(hist_v.at[pl.ds(0, HBINS)], out_hbm.at[pl.ds(wid * HBINS, HBINS)])


_sc_hist = functools.partial(
    pl.kernel,
    mesh=plsc.VectorSubcoreMesh(core_axis_name="c", subcore_axis_name="s"),
    out_type=jax.ShapeDtypeStruct((NWORK * HBINS,), jnp.float32),
    scratch_types=[
        pltpu.VMEM((CHUNK,), jnp.int32),
        pltpu.VMEM((CHUNK,), jnp.int32),
        pltpu.VMEM((16 * HBINS,), jnp.float32),
    ],
    compiler_params=pltpu.CompilerParams(needs_layout_passes=False),
)(_sc_hist_kernel)


def _conv_seg_kernel(off_ref, x_ref, seg_ref, w1_ref, w2_ref,
                     cnn_ref, segg_ref, acc_ref):
    t = pl.program_id(1)
    x = x_ref[0]
    seg = seg_ref[0]
    feats = jax.nn.relu(
        jax.lax.dot_general(w1_ref[...], x, (((0,), (0,)), ((), ()))))
    cnn_ref[0] = jax.lax.dot_general(
        w2_ref[...], feats, (((0,), (0,)), ((), ())))
    segg_ref[0] = seg + off_ref[0]

    @pl.when(t == 0)
    def _init():
        acc_ref[0] = jnp.zeros_like(acc_ref[0])

    m = (jax.lax.broadcasted_iota(jnp.int32, (NTOK, P), 0)
         == seg).astype(jnp.bfloat16)
    upd = jax.lax.dot_general(feats.astype(jnp.bfloat16), m,
                              (((1,), (1,)), ((), ())),
                              preferred_element_type=jnp.float32)
    acc_ref[0] += upd


def _attn_kernel(nt_ref, acc_ref, meta_ref, wq_ref, wk_ref, wv_ref, wo_ref,
                 tl_ref, lab_ref, mask_ref):
    mw = meta_ref[0]
    msum = jnp.sum(mw, axis=0)
    clc = jnp.transpose(msum, (1, 0))
    counts = jnp.sum(clc, axis=0, keepdims=True)
    tokens_t = acc_ref[0] / jnp.maximum(counts, 1.0)
    mx = jnp.max(clc, axis=0, keepdims=True)
    idxv = jax.lax.broadcasted_iota(jnp.int32, (NCLS, NTOK), 0).astype(jnp.float32)
    lab_ref[0] = jnp.min(jnp.where(clc == mx, idxv, jnp.float32(NCLS)),
                         axis=0, keepdims=True)
    n = nt_ref[0]
    valid_row = (jax.lax.broadcasted_iota(jnp.int32, (1, MAXLEN), 1)
                 < n).astype(jnp.float32)
    valid_col = (jax.lax.broadcasted_iota(jnp.int32, (MAXLEN, 1), 0)
                 < n).astype(jnp.float32)
    mask_ref[0] = valid_row

    cdims = (((0,), (0,)), ((), ()))
    q = jax.lax.dot_general(tokens_t, wq_ref[...], cdims)
    k = jax.lax.dot_general(tokens_t, wk_ref[...], cdims)
    v = jax.lax.dot_general(tokens_t, wv_ref[...], cdims)
    scores = jax.lax.dot_general(
        q, k, (((1,), (1,)), ((), ()))) / jnp.sqrt(jnp.float32(CF))
    smax = jnp.max(scores, axis=1, keepdims=True)
    e = jnp.exp(scores - smax)
    attn = e / jnp.sum(e, axis=1, keepdims=True)
    attn = attn * valid_row * valid_col
    ctx = jax.lax.dot_general(attn, v, (((1,), (0,)), ((), ())))
    tl_ref[0] = jax.lax.dot_general(ctx, wo_ref[...], (((1,), (0,)), ((), ())))


def kernel(img, gts, segments, n_tokens, W1, W2, Wq, Wk, Wv, Wo):
    x = img.reshape(B, CIN, N)
    seg3 = segments.reshape(B, 1, N)
    offsets = jnp.concatenate(
        [jnp.zeros((1,), dtype=n_tokens.dtype), jnp.cumsum(n_tokens)[:-1]])
    off1 = (offsets + 1).astype(jnp.int32).reshape(B, 1, 1)

    meta_flat = _sc_hist(segments.reshape(-1), gts.reshape(-1).astype(jnp.int32))
    meta4 = meta_flat.reshape(B, NWORK // B, MAXLEN, NCLS)

    cnn_flat, seg_global_flat, acc = pl.pallas_call(
        _conv_seg_kernel,
        grid=(B, NT),
        in_specs=[
            pl.BlockSpec((1, 1, 1), lambda b, t: (b, 0, 0)),
            pl.BlockSpec((1, CIN, P), lambda b, t: (b, 0, t)),
            pl.BlockSpec((1, 1, P), lambda b, t: (b, 0, t)),
            pl.BlockSpec((CIN, CF), lambda b, t: (0, 0)),
            pl.BlockSpec((CF, NCLS), lambda b, t: (0, 0)),
        ],
        out_specs=[
            pl.BlockSpec((1, NCLS, P), lambda b, t: (b, 0, t)),
            pl.BlockSpec((1, 1, P), lambda b, t: (b, 0, t)),
            pl.BlockSpec((1, CF, NTOK), lambda b, t: (b, 0, 0)),
        ],
        out_shape=[
            jax.ShapeDtypeStruct((B, NCLS, N), jnp.float32),
            jax.ShapeDtypeStruct((B, 1, N), jnp.int32),
            jax.ShapeDtypeStruct((B, CF, NTOK), jnp.float32),
        ],
        compiler_params=pltpu.CompilerParams(
            dimension_semantics=("parallel", "arbitrary")),
    )(off1, x, seg3, W1, W2)

    nt3 = n_tokens.astype(jnp.int32).reshape(B, 1, 1)
    trans_logits, super_labels, mask = pl.pallas_call(
        _attn_kernel,
        grid=(B,),
        in_specs=[
            pl.BlockSpec((1, 1, 1), lambda b: (b, 0, 0)),
            pl.BlockSpec((1, CF, NTOK), lambda b: (b, 0, 0)),
            pl.BlockSpec((1, NWORK // B, MAXLEN, NCLS),
                         lambda b: (b, 0, 0, 0)),
            pl.BlockSpec((CF, CF), lambda b: (0, 0)),
            pl.BlockSpec((CF, CF), lambda b: (0, 0)),
            pl.BlockSpec((CF, CF), lambda b: (0, 0)),
            pl.BlockSpec((CF, NCLS), lambda b: (0, 0)),
        ],
        out_specs=[
            pl.BlockSpec((1, MAXLEN, NCLS), lambda b: (b, 0, 0)),
            pl.BlockSpec((1, 1, MAXLEN), lambda b: (b, 0, 0)),
            pl.BlockSpec((1, 1, MAXLEN), lambda b: (b, 0, 0)),
        ],
        out_shape=[
            jax.ShapeDtypeStruct((B, MAXLEN, NCLS), jnp.float32),
            jax.ShapeDtypeStruct((B, 1, MAXLEN), jnp.float32),
            jax.ShapeDtypeStruct((B, 1, MAXLEN), jnp.float32),
        ],
    )(nt3, acc, meta4, Wq, Wk, Wv, Wo)

    cnn_logits = cnn_flat.reshape(B, NCLS, H, W_)
    seg_global = seg_global_flat.reshape(B, H, W_)
    tokens_ids = jnp.arange(1, B * NTOK + 1, dtype=jnp.int32)
    return (cnn_logits, trans_logits, super_labels.reshape(B, MAXLEN),
            mask.reshape(B, MAXLEN), tokens_ids, seg_global)


# speedup vs baseline: 1.0001x; 1.0001x over previous
import functools

import jax
import jax.numpy as jnp
from jax import lax
from jax.experimental import pallas as pl
from jax.experimental.pallas import tpu as pltpu
from jax.experimental.pallas import tpu_sc as plsc

B, H, W_ = 4, 384, 384
CIN, CF, NCLS = 3, 96, 10
MAXLEN = 512
NTOK = 512
N = H * W_
P = 8192
NT = N // P

SC_NC, SC_NS = 2, 16
NWORK = SC_NC * SC_NS
CHUNK = B * N // NWORK
HBINS = NTOK * NCLS


def _sc_hist_kernel(seg_hbm, gts_hbm, out_hbm, seg_v, gts_v, hist_v):
    wid = lax.axis_index("s") * SC_NC + lax.axis_index("c")
    base = wid * CHUNK
    pltpu.sync_copy(seg_hbm.at[pl.ds(base, CHUNK)], seg_v)
    pltpu.sync_copy(gts_hbm.at[pl.ds(base, CHUNK)], gts_v)

    zeros = jnp.zeros((16,), jnp.float32)

    def zbody(i, carry):
        hist_v[pl.ds(i * 16, 16)] = zeros
        return carry

    lax.fori_loop(0, HBINS // 16, zbody, 0)

    ones = jnp.ones((16,), jnp.float32)

    def body(i, carry):
        for u in range(4):
            s = seg_v[pl.ds((i * 4 + u) * 16, 16)]
            g = gts_v[pl.ds((i * 4 + u) * 16, 16)]
            idx = s * NCLS + g
            plsc.addupdate_scatter(hist_v, [idx], ones)
        return carry

    lax.fori_loop(0, CHUNK // 64, body, 0)
    pltpu.sync_copy(hist_v.at[pl.ds(0, HBINS)], out_hbm.at[pl.ds(wid * HBINS, HBINS)])


_sc_hist = functools.partial(
    pl.kernel,
    mesh=plsc.VectorSubcoreMesh(core_axis_name="c", subcore_axis_name="s"),
    out_type=jax.ShapeDtypeStruct((NWORK * HBINS,), jnp.float32),
    scratch_types=[
        pltpu.VMEM((CHUNK,), jnp.int32),
        pltpu.VMEM((CHUNK,), jnp.int32),
        pltpu.VMEM((HBINS,), jnp.float32),
    ],
    compiler_params=pltpu.CompilerParams(needs_layout_passes=False),
)(_sc_hist_kernel)


def _conv_seg_kernel(off_ref, x_ref, seg_ref, w1_ref, w2_ref,
                     cnn_ref, segg_ref, acc_ref):
    t = pl.program_id(1)
    x = x_ref[0]
    seg = seg_ref[0]
    feats = jax.nn.relu(
        jax.lax.dot_general(w1_ref[...], x, (((0,), (0,)), ((), ()))))
    cnn_ref[0] = jax.lax.dot_general(
        w2_ref[...], feats, (((0,), (0,)), ((), ())))
    segg_ref[0] = seg + off_ref[0]

    @pl.when(t == 0)
    def _init():
        acc_ref[0] = jnp.zeros_like(acc_ref[0])

    m = (jax.lax.broadcasted_iota(jnp.int32, (NTOK, P), 0)
         == seg).astype(jnp.bfloat16)
    upd = jax.lax.dot_general(feats.astype(jnp.bfloat16), m,
                              (((1,), (1,)), ((), ())),
                              preferred_element_type=jnp.float32)
    acc_ref[0] += upd


def _attn_kernel(nt_ref, acc_ref, meta_ref, wq_ref, wk_ref, wv_ref, wo_ref,
                 tl_ref, lab_ref, mask_ref):
    mw = meta_ref[0]
    msum = jnp.sum(mw, axis=0)
    clc = jnp.transpose(msum, (1, 0))
    counts = jnp.sum(clc, axis=0, keepdims=True)
    tokens_t = acc_ref[0] / jnp.maximum(counts, 1.0)
    mx = jnp.max(clc, axis=0, keepdims=True)
    idxv = jax.lax.broadcasted_iota(jnp.int32, (NCLS, NTOK), 0).astype(jnp.float32)
    lab_ref[0] = jnp.min(jnp.where(clc == mx, idxv, jnp.float32(NCLS)),
                         axis=0, keepdims=True)
    n = nt_ref[0]
    valid_row = (jax.lax.broadcasted_iota(jnp.int32, (1, MAXLEN), 1)
                 < n).astype(jnp.float32)
    valid_col = (jax.lax.broadcasted_iota(jnp.int32, (MAXLEN, 1), 0)
                 < n).astype(jnp.float32)
    mask_ref[0] = valid_row

    cdims = (((0,), (0,)), ((), ()))
    q = jax.lax.dot_general(tokens_t, wq_ref[...], cdims)
    k = jax.lax.dot_general(tokens_t, wk_ref[...], cdims)
    v = jax.lax.dot_general(tokens_t, wv_ref[...], cdims)
    scores = jax.lax.dot_general(
        q, k, (((1,), (1,)), ((), ()))) / jnp.sqrt(jnp.float32(CF))
    smax = jnp.max(scores, axis=1, keepdims=True)
    e = jnp.exp(scores - smax)
    attn = e / jnp.sum(e, axis=1, keepdims=True)
    attn = attn * valid_row * valid_col
    ctx = jax.lax.dot_general(attn, v, (((1,), (0,)), ((), ())))
    tl_ref[0] = jax.lax.dot_general(ctx, wo_ref[...], (((1,), (0,)), ((), ())))


def kernel(img, gts, segments, n_tokens, W1, W2, Wq, Wk, Wv, Wo):
    x = img.reshape(B, CIN, N)
    seg3 = segments.reshape(B, 1, N)
    offsets = jnp.concatenate(
        [jnp.zeros((1,), dtype=n_tokens.dtype), jnp.cumsum(n_tokens)[:-1]])
    off1 = (offsets + 1).astype(jnp.int32).reshape(B, 1, 1)

    meta_flat = _sc_hist(segments.reshape(-1), gts.reshape(-1).astype(jnp.int32))
    meta4 = meta_flat.reshape(B, NWORK // B, MAXLEN, NCLS)

    cnn_flat, seg_global_flat, acc = pl.pallas_call(
        _conv_seg_kernel,
        grid=(B, NT),
        in_specs=[
            pl.BlockSpec((1, 1, 1), lambda b, t: (b, 0, 0)),
            pl.BlockSpec((1, CIN, P), lambda b, t: (b, 0, t)),
            pl.BlockSpec((1, 1, P), lambda b, t: (b, 0, t)),
            pl.BlockSpec((CIN, CF), lambda b, t: (0, 0)),
            pl.BlockSpec((CF, NCLS), lambda b, t: (0, 0)),
        ],
        out_specs=[
            pl.BlockSpec((1, NCLS, P), lambda b, t: (b, 0, t)),
            pl.BlockSpec((1, 1, P), lambda b, t: (b, 0, t)),
            pl.BlockSpec((1, CF, NTOK), lambda b, t: (b, 0, 0)),
        ],
        out_shape=[
            jax.ShapeDtypeStruct((B, NCLS, N), jnp.float32),
            jax.ShapeDtypeStruct((B, 1, N), jnp.int32),
            jax.ShapeDtypeStruct((B, CF, NTOK), jnp.float32),
        ],
        compiler_params=pltpu.CompilerParams(
            dimension_semantics=("parallel", "arbitrary")),
    )(off1, x, seg3, W1, W2)

    nt3 = n_tokens.astype(jnp.int32).reshape(B, 1, 1)
    trans_logits, super_labels, mask = pl.pallas_call(
        _attn_kernel,
        grid=(B,),
        in_specs=[
            pl.BlockSpec((1, 1, 1), lambda b: (b, 0, 0)),
            pl.BlockSpec((1, CF, NTOK), lambda b: (b, 0, 0)),
            pl.BlockSpec((1, NWORK // B, MAXLEN, NCLS),
                         lambda b: (b, 0, 0, 0)),
            pl.BlockSpec((CF, CF), lambda b: (0, 0)),
            pl.BlockSpec((CF, CF), lambda b: (0, 0)),
            pl.BlockSpec((CF, CF), lambda b: (0, 0)),
            pl.BlockSpec((CF, NCLS), lambda b: (0, 0)),
        ],
        out_specs=[
            pl.BlockSpec((1, MAXLEN, NCLS), lambda b: (b, 0, 0)),
            pl.BlockSpec((1, 1, MAXLEN), lambda b: (b, 0, 0)),
            pl.BlockSpec((1, 1, MAXLEN), lambda b: (b, 0, 0)),
        ],
        out_shape=[
            jax.ShapeDtypeStruct((B, MAXLEN, NCLS), jnp.float32),
            jax.ShapeDtypeStruct((B, 1, MAXLEN), jnp.float32),
            jax.ShapeDtypeStruct((B, 1, MAXLEN), jnp.float32),
        ],
    )(nt3, acc, meta4, Wq, Wk, Wv, Wo)

    cnn_logits = cnn_flat.reshape(B, NCLS, H, W_)
    seg_global = seg_global_flat.reshape(B, H, W_)
    tokens_ids = jnp.arange(1, B * NTOK + 1, dtype=jnp.int32)
    return (cnn_logits, trans_logits, super_labels.reshape(B, MAXLEN),
            mask.reshape(B, MAXLEN), tokens_ids, seg_global)
